# baseline (speedup 1.0000x reference)
import functools

import jax
import jax.numpy as jnp
from jax import lax
from jax.experimental import pallas as pl
from jax.experimental.pallas import tpu as pltpu
from jax.experimental.pallas import tpu_sc as plsc

_N_H = 8
_N = 64
_NUM_CLS = 117
_NUM_OBJ = 80
_HUMAN = 49
_DIM = 300
_PAIRS = _N_H * _N - _N_H
_REP = 9
_NCHUNK = _PAIRS // _REP
_NBW = 24
_NW = 32


def _l2n(x):
    return x / jnp.maximum(jnp.sqrt(jnp.sum(x * x, axis=-1, keepdims=True)),
                           1e-12)


def _prep_body(lab_ref, ent_ref, rel_ref, nv_ref, oh_ref,
               hp_o, rn_o, wn_o, tp_o, s_o):
    lab = jnp.where(lax.broadcasted_iota(jnp.int32, (_N, 1), 0) < _N_H,
                    _HUMAN, lab_ref[...])
    oh64 = (lab == lax.broadcasted_iota(jnp.int32, (_N, _NUM_OBJ), 1)
            ).astype(jnp.float32)
    ent = ent_ref[...]
    tn = _l2n(jnp.dot(oh64, ent, preferred_element_type=jnp.float32))
    hn = _l2n(ent[_HUMAN:_HUMAN + 1, :])
    wn = _l2n(nv_ref[...])
    rn = _l2n(rel_ref[...])
    hp = hn - jnp.sum(hn * wn, axis=-1, keepdims=True) * wn
    rep = (_REP, _NUM_CLS, _DIM)
    hp_o[...] = jnp.broadcast_to(hp[None], rep)
    rn_o[...] = jnp.broadcast_to(rn[None], rep)
    wn_o[...] = jnp.broadcast_to(wn[None], rep)
    d = lax.dot_general(tn, wn, (((1,), (1,)), ((), ())),
                        preferred_element_type=jnp.float32)
    tp = tn[:, None, :] - d[:, :, None] * wn[None, :, :]
    tp_o[...] = tp
    diff = (hp + rn)[None, :, :] - tp
    s = jnp.sqrt(jnp.sum(diff * diff, axis=-1))
    s_o[...] = jnp.dot(oh_ref[...], s, preferred_element_type=jnp.float32)


def _prep(box_labels, ent_emb, rel_emb, norm_vec, oh504):
    rep = jax.ShapeDtypeStruct((_REP, _NUM_CLS, _DIM), jnp.float32)
    return pl.pallas_call(
        _prep_body,
        out_shape=(rep, rep, rep,
                   jax.ShapeDtypeStruct((_N, _NUM_CLS, _DIM), jnp.float32),
                   jax.ShapeDtypeStruct((_PAIRS, _NUM_CLS), jnp.float32)),
    )(box_labels.reshape(_N, 1), ent_emb, rel_emb, norm_vec, oh504)


_BIG = jax.ShapeDtypeStruct((_PAIRS, _NUM_CLS, _DIM), jnp.float32)


@functools.partial(
    pl.kernel,
    out_type=[_BIG, _BIG, _BIG, _BIG],
    mesh=plsc.VectorSubcoreMesh(core_axis_name="c", subcore_axis_name="s"),
    scratch_types=[
        pltpu.VMEM_SHARED((3, _REP, _NUM_CLS, _DIM), jnp.float32),
        pltpu.SemaphoreType.DMA,
    ],
)
def _expand(hp_hbm, rn_hbm, wn_hbm, tp_hbm,
            h_out, r_out, w_out, t_out, tab_s, sem):
    cid = lax.axis_index("c")
    sid = lax.axis_index("s")
    wid = sid * 2 + cid

    @pl.when(sid == 0)
    def _load():
        pltpu.sync_copy(hp_hbm, tab_s.at[0])
        pltpu.sync_copy(rn_hbm, tab_s.at[1])
        pltpu.sync_copy(wn_hbm, tab_s.at[2])

    plsc.subcore_barrier()

    outs = (h_out, r_out, w_out)
    for w in range(_NBW):
        @pl.when(wid == w)
        def _bcast(w=w):
            descs = []
            for i in range(7):
                task = w * 7 + i
                m, c = divmod(task, _NCHUNK)
                descs.append(pltpu.async_copy(
                    tab_s.at[m], outs[m].at[pl.ds(c * _REP, _REP)], sem))
            for dsc in descs:
                dsc.wait()

    for x in range(_N_H):
        @pl.when(wid == _NBW + x)
        def _trun(x=x):
            descs = []
            if x > 0:
                descs.append(pltpu.async_copy(
                    tp_hbm.at[pl.ds(0, x)],
                    t_out.at[pl.ds(63 * x, x)], sem))
            descs.append(pltpu.async_copy(
                tp_hbm.at[pl.ds(x + 1, 63 - x)],
                t_out.at[pl.ds(63 * x + x, 63 - x)], sem))
            for dsc in descs:
                dsc.wait()


def _static_onehot():
    import numpy as np
    ys = np.array([j + (1 if j >= x else 0)
                   for x in range(_N_H) for j in range(_N - 1)], np.int32)
    return (ys[:, None] == np.arange(_N)[None, :]).astype(np.float32)


_OH504 = _static_onehot()


def kernel(box_labels, ent_emb, rel_emb, norm_vec):
    hp, rn, wn, tp, scores = _prep(box_labels, ent_emb, rel_emb, norm_vec,
                                   jnp.asarray(_OH504))
    h_keep, r_keep, w_keep, t_keep = _expand(hp, rn, wn, tp)
    return (h_keep, r_keep, w_keep, t_keep, scores)


# speedup vs baseline: 6.7349x; 6.7349x over previous
import functools

import jax
import jax.numpy as jnp
from jax import lax
from jax.experimental import pallas as pl
from jax.experimental.pallas import tpu as pltpu
from jax.experimental.pallas import tpu_sc as plsc

_N_H = 8
_N = 64
_NUM_CLS = 117
_NUM_OBJ = 80
_HUMAN = 49
_DIM = 300
_PAIRS = _N_H * _N - _N_H
_DEPTH = 16


def _l2n(x):
    return x / jnp.maximum(jnp.sqrt(jnp.sum(x * x, axis=-1, keepdims=True)),
                           1e-12)


def _prep_body(lab_ref, ent_ref, rel_ref, nv_ref, oh_ref,
               hp_o, rn_o, wn_o, tp_o, s_o):
    lab = jnp.where(lax.broadcasted_iota(jnp.int32, (_N, 1), 0) < _N_H,
                    _HUMAN, lab_ref[...])
    oh64 = (lab == lax.broadcasted_iota(jnp.int32, (_N, _NUM_OBJ), 1)
            ).astype(jnp.float32)
    ent = ent_ref[...]
    tn = _l2n(jnp.dot(oh64, ent, preferred_element_type=jnp.float32))
    hn = _l2n(ent[_HUMAN:_HUMAN + 1, :])
    wn = _l2n(nv_ref[...])
    rn = _l2n(rel_ref[...])
    hp = hn - jnp.sum(hn * wn, axis=-1, keepdims=True) * wn
    hp_o[...] = hp
    rn_o[...] = rn
    wn_o[...] = wn
    d = lax.dot_general(tn, wn, (((1,), (1,)), ((), ())),
                        preferred_element_type=jnp.float32)
    tp = tn[:, None, :] - d[:, :, None] * wn[None, :, :]
    tp_o[...] = tp
    diff = (hp + rn)[None, :, :] - tp
    s = jnp.sqrt(jnp.sum(diff * diff, axis=-1))
    s_o[...] = jnp.dot(oh_ref[...], s, preferred_element_type=jnp.float32)


def _prep(box_labels, ent_emb, rel_emb, norm_vec, oh504):
    small = jax.ShapeDtypeStruct((_NUM_CLS, _DIM), jnp.float32)
    return pl.pallas_call(
        _prep_body,
        out_shape=(small, small, small,
                   jax.ShapeDtypeStruct((_N, _NUM_CLS, _DIM), jnp.float32),
                   jax.ShapeDtypeStruct((_PAIRS, _NUM_CLS), jnp.float32)),
    )(box_labels.reshape(_N, 1), ent_emb, rel_emb, norm_vec, oh504)


_BIG = jax.ShapeDtypeStruct((_PAIRS, _NUM_CLS, _DIM), jnp.float32)


def _t_scatter(buf, y, t_out, sem):
    def fire(x, c):
        j = jnp.where(y < x, y, y - 1)

        @pl.when(x != y)
        def _():
            pltpu.async_copy(buf, t_out.at[63 * x + j], sem)
        return c

    lax.fori_loop(0, _N_H, fire, 0)

    def drain(x, c):
        j = jnp.where(y < x, y, y - 1)

        @pl.when(x != y)
        def _():
            pltpu.make_async_copy(buf, t_out.at[63 * x + j], sem).wait()
        return c

    lax.fori_loop(0, _N_H, drain, 0)


@functools.partial(
    pl.kernel,
    out_type=[_BIG, _BIG, _BIG, _BIG],
    mesh=plsc.VectorSubcoreMesh(core_axis_name="c", subcore_axis_name="s"),
    scratch_types=[
        pltpu.VMEM((_NUM_CLS, _DIM), jnp.float32),
        pltpu.VMEM((_NUM_CLS, _DIM), jnp.float32),
        pltpu.SemaphoreType.DMA,
        pltpu.SemaphoreType.DMA,
    ],
)
def _expand(hp_hbm, rn_hbm, wn_hbm, tp_hbm,
            h_out, r_out, w_out, t_out, tab_v, aux_v, sem, lsem):
    cid = lax.axis_index("c")
    sid = lax.axis_index("s")
    wid = sid * 2 + cid
    y0 = 2 * wid
    y1 = y0 + 1

    pltpu.async_copy(tp_hbm.at[y0], aux_v, lsem)

    outs = (h_out, r_out, w_out)
    tabs = (hp_hbm, rn_hbm, wn_hbm)
    bases = (0, 11, 22)
    counts = (11, 11, 10)

    for m in range(3):
        @pl.when(jnp.logical_and(wid >= bases[m], wid < bases[m] + counts[m]))
        def _bcast(m=m):
            pltpu.sync_copy(tabs[m], tab_v)
            idx = wid - bases[m]
            ra = idx * _PAIRS // counts[m]
            rb = (idx + 1) * _PAIRS // counts[m]

            def fire(r, c):
                pltpu.async_copy(tab_v, outs[m].at[r], sem)

                @pl.when(r - ra >= _DEPTH)
                def _():
                    pltpu.make_async_copy(tab_v, outs[m].at[r - _DEPTH],
                                          sem).wait()
                return c

            lax.fori_loop(ra, rb, fire, 0)

            def drain(r, c):
                pltpu.make_async_copy(tab_v, outs[m].at[r], sem).wait()
                return c

            lax.fori_loop(jnp.maximum(ra, rb - _DEPTH), rb, drain, 0)

    pltpu.make_async_copy(tp_hbm.at[y0], aux_v, lsem).wait()
    _t_scatter(aux_v, y0, t_out, sem)
    pltpu.sync_copy(tp_hbm.at[y1], tab_v)
    _t_scatter(tab_v, y1, t_out, sem)


def _static_onehot():
    import numpy as np
    ys = np.array([j + (1 if j >= x else 0)
                   for x in range(_N_H) for j in range(_N - 1)], np.int32)
    return (ys[:, None] == np.arange(_N)[None, :]).astype(np.float32)


_OH504 = _static_onehot()


def kernel(box_labels, ent_emb, rel_emb, norm_vec):
    hp, rn, wn, tp, scores = _prep(box_labels, ent_emb, rel_emb, norm_vec,
                                   jnp.asarray(_OH504))
    h_keep, r_keep, w_keep, t_keep = _expand(hp, rn, wn, tp)
    return (h_keep, r_keep, w_keep, t_keep, scores)
